# baseline (speedup 1.0000x reference)
import math

import jax
import jax.numpy as jnp
from jax.experimental import pallas as pl
from jax.experimental.pallas import tpu as pltpu

_N = 1024
_NFEAT = 128
_HIDDEN = 64
_NCLASS = 40
_NUM_LAYERS = 2
_ALPHA = 0.1
_THETA = 0.5
_K = 8
_BLK = _N // _K


def _split_cat(hf):
    hb = hf.astype(jnp.bfloat16)
    hr = (hf - hb.astype(jnp.float32)).astype(jnp.bfloat16)
    return jnp.concatenate([hb, hr], axis=1)


def _mm(a, b):
    return jax.lax.dot_general(a, b, (((1,), (0,)), ((), ())),
                               precision=jax.lax.Precision.HIGHEST)


def _magg(maskb, hcat):
    o = jax.lax.dot_general(maskb, hcat, (((0,), (0,)), ((), ())),
                            preferred_element_type=jnp.float32)
    return o


def _gcn2_fwd(x_ref, adj_ref, w0_ref, b0_ref, w1_ref, b1_ref, cw_ref,
              out_ref, maskb_ref, z_ref):
    i = pl.program_id(0)

    adj = adj_ref[...]
    nz = (adj != 0.0)
    maskb_ref[pl.ds(i * _BLK, _BLK), :] = nz.astype(jnp.bfloat16)
    zeros_here = jnp.float32(_BLK * _N) - jnp.sum(nz.astype(jnp.float32))

    @pl.when(i == 0)
    def _init():
        z_ref[...] = jnp.zeros_like(z_ref)

    z_ref[...] = z_ref[...] + zeros_here

    @pl.when(i == _K - 1)
    def _epilogue():
        h = jnp.maximum(_mm(x_ref[...], w0_ref[...]) + b0_ref[...], 0.0)
        x0 = h
        z = z_ref[0, 0]
        maskb = maskb_ref[...]
        row_is0 = jax.lax.broadcasted_iota(jnp.int32, (_N, 1), 0) == 0

        for layer in range(_NUM_LAYERS):
            beta = math.log(_THETA / (layer + 1) + 1.0)
            o = _magg(maskb, _split_cat(h))
            agg = o[:, :_HIDDEN] + o[:, _HIDDEN:]
            agg = agg + jnp.where(row_is0, z * h[0:1, :], 0.0)
            out = agg * (1.0 - _ALPHA) + _ALPHA * x0
            out = (1.0 - beta) * out + beta * _mm(out, cw_ref[layer])
            h = jnp.maximum(out, 0.0)

        logits = _mm(h, w1_ref[...]) + b1_ref[...]
        m = jnp.max(logits, axis=-1, keepdims=True)
        s = logits - m
        lse = jnp.log(jnp.sum(jnp.exp(s), axis=-1, keepdims=True))
        out_ref[...] = s - lse


def kernel(x, adj_t, lin0_w, lin0_b, lin1_w, lin1_b, conv_w):
    b0 = lin0_b.reshape(1, _HIDDEN)
    b1 = lin1_b.reshape(1, _NCLASS)
    full = lambda *shape: pl.BlockSpec(shape, lambda i: tuple(0 for _ in shape))
    return pl.pallas_call(
        _gcn2_fwd,
        grid=(_K,),
        in_specs=[
            full(_N, _NFEAT),
            pl.BlockSpec((_BLK, _N), lambda i: (i, 0)),
            full(_NFEAT, _HIDDEN),
            full(1, _HIDDEN),
            full(_HIDDEN, _NCLASS),
            full(1, _NCLASS),
            full(_NUM_LAYERS, _HIDDEN, _HIDDEN),
        ],
        out_specs=full(_N, _NCLASS),
        out_shape=jax.ShapeDtypeStruct((_N, _NCLASS), jnp.float32),
        scratch_shapes=[
            pltpu.VMEM((_N, _N), jnp.bfloat16),
            pltpu.VMEM((1, 1), jnp.float32),
        ],
    )(x, adj_t, lin0_w, b0, lin1_w, b1, conv_w)


# speedup vs baseline: 1.2931x; 1.2931x over previous
import math

import jax
import jax.numpy as jnp
from jax.experimental import pallas as pl

_N = 1024
_NFEAT = 128
_HIDDEN = 64
_NCLASS = 40
_NUM_LAYERS = 2
_ALPHA = 0.1
_THETA = 0.5


def _gcn2_fwd(x_ref, adj_ref, w0_ref, b0_ref, w1_ref, b1_ref, cw_ref, out_ref):
    hi = jax.lax.Precision.HIGHEST

    def mm(a, b, dims):
        return jax.lax.dot_general(a, b, (dims, ((), ())), precision=hi)

    x = x_ref[...]
    h = jnp.maximum(mm(x, w0_ref[...], ((1,), (0,))) + b0_ref[...], 0.0)
    x0 = h

    adj = adj_ref[...]
    mask = (adj != 0.0).astype(jnp.float32)
    z = jnp.float32(_N * _N) - jnp.sum(mask)
    maskb = mask.astype(jnp.bfloat16)
    row_is0 = jax.lax.broadcasted_iota(jnp.int32, (_N, 1), 0) == 0

    def masked_agg(hf):
        hb = hf.astype(jnp.bfloat16)
        hr = (hf - hb.astype(jnp.float32)).astype(jnp.bfloat16)
        hcat = jnp.concatenate([hb, hr], axis=1)
        o = jax.lax.dot_general(maskb, hcat, ((((0,), (0,))), ((), ())),
                                preferred_element_type=jnp.float32)
        return o[:, :_HIDDEN] + o[:, _HIDDEN:]

    for layer in range(_NUM_LAYERS):
        beta = math.log(_THETA / (layer + 1) + 1.0)
        agg = masked_agg(h)
        agg = agg + jnp.where(row_is0, z * h[0:1, :], 0.0)
        out = agg * (1.0 - _ALPHA) + _ALPHA * x0
        out = (1.0 - beta) * out + beta * mm(out, cw_ref[layer], ((1,), (0,)))
        h = jnp.maximum(out, 0.0)

    logits = mm(h, w1_ref[...], ((1,), (0,))) + b1_ref[...]
    m = jnp.max(logits, axis=-1, keepdims=True)
    s = logits - m
    lse = jnp.log(jnp.sum(jnp.exp(s), axis=-1, keepdims=True))
    out_ref[...] = s - lse


def kernel(x, adj_t, lin0_w, lin0_b, lin1_w, lin1_b, conv_w):
    b0 = lin0_b.reshape(1, _HIDDEN)
    b1 = lin1_b.reshape(1, _NCLASS)
    return pl.pallas_call(
        _gcn2_fwd,
        out_shape=jax.ShapeDtypeStruct((_N, _NCLASS), jnp.float32),
    )(x, adj_t, lin0_w, b0, lin1_w, b1, conv_w)
